# baseline (speedup 1.0000x reference)
import functools

import jax
import jax.numpy as jnp
from jax.experimental import pallas as pl
from jax.experimental.pallas import tpu as pltpu
from jax.experimental.pallas import tpu_sc as plsc

_POSITION = 0


@functools.partial(jax.jit, static_argnums=(1,))
def _pool_rows(inputs, position):
    b, _, d = inputs.shape
    mesh = plsc.ScalarSubcoreMesh(axis_name="c", num_cores=1)

    @functools.partial(
        pl.kernel,
        out_type=jax.ShapeDtypeStruct((b, d), jnp.float32),
        mesh=mesh,
        scratch_types=[],
    )
    def pooler(in_hbm, out_hbm):
        pltpu.sync_copy(in_hbm.at[:, position], out_hbm)

    return pooler(inputs)


def kernel(inputs, mask):
    del mask
    return _pool_rows(inputs, _POSITION)
